# baseline (speedup 1.0000x reference)
import functools

import jax
import jax.numpy as jnp
from jax import lax
from jax.experimental import pallas as pl
from jax.experimental.pallas import tpu as pltpu
from jax.experimental.pallas import tpu_sc as plsc

D_MODEL = 1024
SCALE = 32.0

NC = 2
NS = 16
NW = NC * NS
LANES = 16

B_TOTAL = 4 * 2048
SC_ROWS = 4608
TC_ROWS = B_TOTAL - SC_ROWS

RPW = SC_ROWS // NW
CHUNK = 16
NCHUNK = RPW // CHUNK
NBUF = 7
LOOKAHEAD = 4

TCG = 64
TCNB = 4
TCK = 2
NGROUP = TC_ROWS // TCG


def _make_sc_kernel():
    mesh = plsc.VectorSubcoreMesh(core_axis_name="c", subcore_axis_name="s")

    @functools.partial(
        pl.kernel,
        mesh=mesh,
        out_type=jax.ShapeDtypeStruct((SC_ROWS, D_MODEL), jnp.float32),
        scratch_types=(
            [pltpu.VMEM((NCHUNK, CHUNK), jnp.int32)]
            + [pltpu.VMEM((CHUNK, D_MODEL), jnp.float32)] * NBUF
            + [pltpu.SemaphoreType.DMA] * (2 * NBUF)
        ),
    )
    def sc_kernel(x_hbm, table_hbm, out_hbm, idx_v,
                  b0, b1, b2, b3, b4, b5, b6,
                  si0, si1, si2, si3, si4, si5, si6,
                  so0, so1, so2, so3, so4, so5, so6):
        wid = lax.axis_index("s") * NC + lax.axis_index("c")
        base = wid * RPW
        pltpu.sync_copy(x_hbm.at[wid], idx_v)

        bufs = (b0, b1, b2, b3, b4, b5, b6)
        sins = (si0, si1, si2, si3, si4, si5, si6)
        souts = (so0, so1, so2, so3, so4, so5, so6)
        gathers = [None] * NBUF
        outs = [None] * NBUF

        def start_gather(j):
            p = j % NBUF
            gathers[p] = pltpu.async_copy(
                table_hbm.at[idx_v.at[j]], bufs[p], sins[p])

        for j in range(min(LOOKAHEAD, NCHUNK)):
            start_gather(j)

        for g in range(NCHUNK):
            p = g % NBUF
            j = g + LOOKAHEAD
            if j < NCHUNK:
                if j - NBUF >= 0 and outs[j % NBUF] is not None:
                    outs[j % NBUF].wait()
                    outs[j % NBUF] = None
                start_gather(j)
            gathers[p].wait()

            buf = bufs[p]

            def scale_row(r, carry, buf=buf):
                for col in range(D_MODEL // LANES):
                    sl = pl.ds(col * LANES, LANES)
                    buf[r, sl] = buf[r, sl] * SCALE
                return carry

            lax.fori_loop(0, CHUNK, scale_row, 0)

            outs[p] = pltpu.async_copy(
                buf, out_hbm.at[pl.ds(base + g * CHUNK, CHUNK)], souts[p])

        for p in range(NBUF):
            if outs[p] is not None:
                outs[p].wait()
                outs[p] = None

    return sc_kernel


def _make_tc_kernel():
    def tc_body(idx_smem, table_hbm, out_hbm,
                b0, b1, b2, b3,
                si0, si1, si2, si3,
                so0, so1, so2, so3):
        bufs = (b0, b1, b2, b3)
        sins = (si0, si1, si2, si3)
        souts = (so0, so1, so2, so3)
        gathers = [None] * TCNB
        outs = [None] * TCNB

        def start_group(j):
            p = j % TCNB
            cps = []
            for r in range(TCG):
                idx = idx_smem[j * TCG + r]
                cps.append(pltpu.make_async_copy(
                    table_hbm.at[idx], bufs[p].at[r], sins[p]))
            for c in cps:
                c.start()
            gathers[p] = cps

        for j in range(min(TCK, NGROUP)):
            start_group(j)

        for g in range(NGROUP):
            p = g % TCNB
            j = g + TCK
            if j < NGROUP:
                if j - TCNB >= 0 and outs[j % TCNB] is not None:
                    outs[j % TCNB].wait()
                    outs[j % TCNB] = None
                start_group(j)
            for c in gathers[p]:
                c.wait()

            bufs[p][...] = bufs[p][...] * SCALE

            outs[p] = pltpu.async_copy(
                bufs[p], out_hbm.at[pl.ds(g * TCG, TCG)], souts[p])

        for p in range(TCNB):
            if outs[p] is not None:
                outs[p].wait()
                outs[p] = None

    return pl.pallas_call(
        tc_body,
        in_specs=[
            pl.BlockSpec(memory_space=pltpu.SMEM),
            pl.BlockSpec(memory_space=pltpu.HBM),
        ],
        out_specs=pl.BlockSpec(memory_space=pltpu.HBM),
        out_shape=jax.ShapeDtypeStruct((TC_ROWS, D_MODEL), jnp.float32),
        scratch_shapes=(
            [pltpu.VMEM((TCG, D_MODEL), jnp.float32)] * TCNB
            + [pltpu.SemaphoreType.DMA] * (2 * TCNB)
        ),
    )


_sc_kernel = _make_sc_kernel()
_tc_kernel = _make_tc_kernel()


def kernel(x, table):
    flat = x.astype(jnp.int32).reshape(-1)
    idx_sc = flat[:SC_ROWS].reshape(NW, NCHUNK, CHUNK)
    idx_tc = flat[SC_ROWS:]
    sc_out = _sc_kernel(idx_sc, table)
    tc_out = _tc_kernel(idx_tc, table)
    out = jnp.concatenate([sc_out, tc_out], axis=0)
    return out.reshape(x.shape + (D_MODEL,))


# speedup vs baseline: 1.5398x; 1.5398x over previous
import functools

import jax
import jax.numpy as jnp
from jax import lax
from jax.experimental import pallas as pl
from jax.experimental.pallas import tpu as pltpu
from jax.experimental.pallas import tpu_sc as plsc

D_MODEL = 1024
SCALE = 32.0

NC = 2
NS = 16
NW = NC * NS
LANES = 16

X_ROWS = 4
X_COLS = 2048
B_TOTAL = X_ROWS * X_COLS
RPW = B_TOTAL // NW
WPR = X_COLS // RPW
CHUNK = 16
NCHUNK = RPW // CHUNK
HALF = CHUNK // 2
NBUF = 7
LOOKAHEAD = 5


def _make_kernel():
    mesh = plsc.VectorSubcoreMesh(core_axis_name="c", subcore_axis_name="s")

    @functools.partial(
        pl.kernel,
        mesh=mesh,
        out_type=jax.ShapeDtypeStruct((B_TOTAL, D_MODEL), jnp.float32),
        scratch_types=(
            [pltpu.VMEM((RPW,), jnp.int32)]
            + [pltpu.VMEM((CHUNK, D_MODEL), jnp.float32)] * NBUF
            + [pltpu.SemaphoreType.DMA] * (2 * NBUF)
        ),
    )
    def emb_kernel(x_hbm, table_hbm, out_hbm, idx_v,
                   b0, b1, b2, b3, b4, b5, b6,
                   si0, si1, si2, si3, si4, si5, si6,
                   so0, so1, so2, so3, so4, so5, so6):
        wid = lax.axis_index("s") * NC + lax.axis_index("c")
        base = wid * RPW
        pltpu.sync_copy(
            x_hbm.at[wid // WPR, pl.ds((wid % WPR) * RPW, RPW)], idx_v)

        bufs = (b0, b1, b2, b3, b4, b5, b6)
        sins = (si0, si1, si2, si3, si4, si5, si6)
        souts = (so0, so1, so2, so3, so4, so5, so6)
        gathers = [None] * NBUF
        outs = [None] * NBUF

        def start_gather(j):
            p = j % NBUF
            gathers[p] = pltpu.async_copy(
                table_hbm.at[idx_v.at[pl.ds(j * CHUNK, CHUNK)]],
                bufs[p], sins[p])

        for j in range(min(LOOKAHEAD, NCHUNK)):
            start_gather(j)

        for g in range(NCHUNK):
            p = g % NBUF
            j = g + LOOKAHEAD
            if j < NCHUNK:
                if j - NBUF >= 0 and outs[j % NBUF] is not None:
                    for c in outs[j % NBUF]:
                        c.wait()
                    outs[j % NBUF] = None
                start_gather(j)
            gathers[p].wait()

            buf = bufs[p]

            def scale_row(r, carry, buf=buf):
                for col in range(D_MODEL // LANES):
                    sl = pl.ds(col * LANES, LANES)
                    buf[r, sl] = buf[r, sl] * SCALE
                return carry

            lax.fori_loop(0, HALF, scale_row, 0)
            cp_a = pltpu.async_copy(
                buf.at[pl.ds(0, HALF)],
                out_hbm.at[pl.ds(base + g * CHUNK, HALF)], souts[p])
            lax.fori_loop(HALF, CHUNK, scale_row, 0)
            cp_b = pltpu.async_copy(
                buf.at[pl.ds(HALF, HALF)],
                out_hbm.at[pl.ds(base + g * CHUNK + HALF, HALF)], souts[p])
            outs[p] = (cp_a, cp_b)

        for p in range(NBUF):
            if outs[p] is not None:
                for c in outs[p]:
                    c.wait()
                outs[p] = None

    return emb_kernel


_emb_kernel = _make_kernel()


def kernel(x, table):
    out = _emb_kernel(x.astype(jnp.int32), table)
    return out.reshape(x.shape + (D_MODEL,))
